# baseline (speedup 1.0000x reference)
import jax
import jax.numpy as jnp
from jax import lax
from jax.experimental import pallas as pl

BATCH = 1024
MAX_ATOMS = 128
N_FEAT = 256
MB = 8


def _body(ds_ref, nf_ref, out_ref):
    x = nf_ref[...]
    counts = ds_ref[:, 0]
    atom_ids = lax.broadcasted_iota(jnp.int32, (MB, MAX_ATOMS), 1)
    mask = (atom_ids < counts[:, None]).astype(x.dtype)
    acc = jnp.sum(x * mask[:, :, None], axis=1)
    out_ref[...] = jnp.maximum(acc, 0.0)


def kernel(node_features, data_slice):
    grid = (BATCH // MB,)
    return pl.pallas_call(
        _body,
        grid=grid,
        in_specs=[
            pl.BlockSpec((MB, 2), lambda i: (i, 0)),
            pl.BlockSpec((MB, MAX_ATOMS, N_FEAT), lambda i: (i, 0, 0)),
        ],
        out_specs=pl.BlockSpec((MB, N_FEAT), lambda i: (i, 0)),
        out_shape=jax.ShapeDtypeStruct((BATCH, N_FEAT), jnp.float32),
    )(data_slice, node_features)


# speedup vs baseline: 1.1480x; 1.1480x over previous
import functools

import jax
import jax.numpy as jnp
from jax import lax
from jax.experimental import pallas as pl
from jax.experimental.pallas import tpu as pltpu
from jax.experimental.pallas import tpu_sc as plsc

BATCH = 1024
MAX_ATOMS = 128
N_FEAT = 256
LANES = 16
CH = 16
NWORKERS = 32
MPW = BATCH // NWORKERS
NSLICE = N_FEAT // LANES


def _sc_body(nf_hbm, cnt_hbm, out_hbm, cnt_v, chunk_v, out_v, sem):
    core = lax.axis_index("c")
    sub = lax.axis_index("s")
    wid = sub * 2 + core
    base = wid * MPW

    pltpu.sync_copy(cnt_hbm.at[pl.ds(base, MPW)], cnt_v.at[pl.ds(0, MPW)])

    def mol_body(m, carry):
        c = cnt_v[pl.ds(m, LANES)][0]
        nch = (c + (CH - 1)) // CH

        def issue_body(j, carry):
            pltpu.make_async_copy(
                nf_hbm.at[base + m, pl.ds(j * CH, CH)],
                chunk_v.at[pl.ds(j * CH, CH)],
                sem,
            ).start()
            return carry

        lax.fori_loop(0, nch, issue_body, 0)

        def drain_body(j, carry):
            pltpu.make_async_copy(
                nf_hbm.at[base + m, pl.ds(0, CH)],
                chunk_v.at[pl.ds(0, CH)],
                sem,
            ).wait()
            return carry

        lax.fori_loop(0, nch, drain_body, 0)

        def row_body(r, acc):
            return tuple(
                acc[k] + chunk_v[r, pl.ds(k * LANES, LANES)]
                for k in range(NSLICE)
            )

        acc0 = tuple(jnp.zeros((LANES,), jnp.float32) for _ in range(NSLICE))
        acc = lax.fori_loop(0, c, row_body, acc0)
        for k in range(NSLICE):
            out_v[m, pl.ds(k * LANES, LANES)] = jnp.maximum(acc[k], 0.0)
        return carry

    lax.fori_loop(0, MPW, mol_body, 0)
    pltpu.sync_copy(out_v, out_hbm.at[pl.ds(base, MPW)])


def kernel(node_features, data_slice):
    counts = data_slice[:, 0]
    mesh = plsc.VectorSubcoreMesh(core_axis_name="c", subcore_axis_name="s")
    f = functools.partial(
        pl.kernel,
        out_type=jax.ShapeDtypeStruct((BATCH, N_FEAT), jnp.float32),
        mesh=mesh,
        scratch_types=[
            pltpu.VMEM((MPW + LANES,), jnp.int32),
            pltpu.VMEM((MAX_ATOMS, N_FEAT), jnp.float32),
            pltpu.VMEM((MPW, N_FEAT), jnp.float32),
            pltpu.SemaphoreType.DMA,
        ],
    )(_sc_body)
    return f(node_features, counts)


# speedup vs baseline: 1.5266x; 1.3298x over previous
import functools

import jax
import jax.numpy as jnp
from jax import lax
from jax.experimental import pallas as pl
from jax.experimental.pallas import tpu as pltpu
from jax.experimental.pallas import tpu_sc as plsc

BATCH = 1024
MAX_ATOMS = 128
N_FEAT = 256
LANES = 16
CH = 16
NWORKERS = 32
MPW = BATCH // NWORKERS
NSLICE = N_FEAT // LANES


def _sc_body(nf_hbm, cnt_hbm, out_hbm, cnt_v, buf0, buf1, out_v, sem0, sem1):
    core = lax.axis_index("c")
    sub = lax.axis_index("s")
    wid = sub * 2 + core
    base = wid * MPW

    pltpu.sync_copy(cnt_hbm.at[pl.ds(base, MPW)], cnt_v.at[pl.ds(0, MPW)])

    def count_of(m):
        return cnt_v[pl.ds(m, LANES)][0]

    def issue(m, buf, sem):
        nch = (count_of(m) + (CH - 1)) // CH

        def issue_body(j, carry):
            pltpu.make_async_copy(
                nf_hbm.at[base + m, pl.ds(j * CH, CH)],
                buf.at[pl.ds(j * CH, CH)],
                sem,
            ).start()
            return carry

        lax.fori_loop(0, nch, issue_body, 0)

    def consume(m, buf, sem):
        c = count_of(m)
        nch = (c + (CH - 1)) // CH

        def drain_body(j, carry):
            pltpu.make_async_copy(
                nf_hbm.at[base + m, pl.ds(0, CH)],
                buf.at[pl.ds(0, CH)],
                sem,
            ).wait()
            return carry

        lax.fori_loop(0, nch, drain_body, 0)

        def row2_body(t, acc):
            r = 2 * t
            return tuple(
                acc[k]
                + buf[r, pl.ds(k * LANES, LANES)]
                + buf[r + 1, pl.ds(k * LANES, LANES)]
                for k in range(NSLICE)
            )

        acc0 = tuple(jnp.zeros((LANES,), jnp.float32) for _ in range(NSLICE))
        acc = lax.fori_loop(0, c // 2, row2_body, acc0)
        odd = (c % 2) == 1
        zero = jnp.zeros((LANES,), jnp.float32)
        acc = tuple(
            acc[k]
            + jnp.where(odd, buf[c - 1, pl.ds(k * LANES, LANES)], zero)
            for k in range(NSLICE)
        )
        for k in range(NSLICE):
            out_v[m, pl.ds(k * LANES, LANES)] = jnp.maximum(acc[k], 0.0)

    issue(0, buf0, sem0)

    def pair_body(t, carry):
        m = 2 * t
        issue(m + 1, buf1, sem1)
        consume(m, buf0, sem0)

        @pl.when(m + 2 < MPW)
        def _():
            issue(m + 2, buf0, sem0)

        consume(m + 1, buf1, sem1)
        return carry

    lax.fori_loop(0, MPW // 2, pair_body, 0)
    pltpu.sync_copy(out_v, out_hbm.at[pl.ds(base, MPW)])


def kernel(node_features, data_slice):
    counts = data_slice[:, 0]
    mesh = plsc.VectorSubcoreMesh(core_axis_name="c", subcore_axis_name="s")
    f = functools.partial(
        pl.kernel,
        out_type=jax.ShapeDtypeStruct((BATCH, N_FEAT), jnp.float32),
        mesh=mesh,
        scratch_types=[
            pltpu.VMEM((MPW + LANES,), jnp.int32),
            pltpu.VMEM((MAX_ATOMS, N_FEAT), jnp.float32),
            pltpu.VMEM((MAX_ATOMS, N_FEAT), jnp.float32),
            pltpu.VMEM((MPW, N_FEAT), jnp.float32),
            pltpu.SemaphoreType.DMA,
            pltpu.SemaphoreType.DMA,
        ],
    )(_sc_body)
    return f(node_features, counts)
